# baseline (speedup 1.0000x reference)
import functools
import jax
import jax.numpy as jnp
from jax import lax
from jax.experimental import pallas as pl
from jax.experimental.pallas import tpu as pltpu
from jax.experimental.pallas import tpu_sc as plsc

_M, _N = 128, 32768
_NC, _NS, _L = 2, 16, 16
_NW = _NC * _NS
_RPW = _M // _NW
_Q = 4
_QLEN = _N // _Q
_UNROLL = 8


def _sc_body(x_hbm, o_hbm, buf0, buf1, si0, si1, so0, so1):
    bufs = (buf0, buf1)
    sins = (si0, si1)
    souts = (so0, so1)
    wid = lax.axis_index("s") * _NC + lax.axis_index("c")
    r0 = wid * _RPW

    def compute(buf):
        def body(i, cs):
            cs = list(cs)
            base = i * (_L * _UNROLL)
            for u in range(_UNROLL):
                for q in range(_Q):
                    off = q * _QLEN + base + u * _L
                    v = buf[0, pl.ds(off, _L)]
                    s = plsc.cumsum(v)
                    buf[0, pl.ds(off, _L)] = s + cs[q]
                    cs[q] = cs[q] + s[_L - 1]
            return tuple(cs)

        totals = lax.fori_loop(0, _QLEN // (_L * _UNROLL), body,
                               (jnp.float32(0),) * _Q)

        offs = []
        acc = totals[0]
        for q in range(1, _Q):
            offs.append(acc)
            acc = acc + totals[q]

        def fix(i, _):
            base = i * (_L * _UNROLL)
            for u in range(_UNROLL):
                for q in range(1, _Q):
                    off = q * _QLEN + base + u * _L
                    buf[0, pl.ds(off, _L)] = buf[0, pl.ds(off, _L)] + offs[q - 1]
            return 0

        lax.fori_loop(0, _QLEN // (_L * _UNROLL), fix, 0)

    def start_in(ri):
        return pltpu.async_copy(
            x_hbm.at[pl.ds(r0 + ri, 1), :], bufs[ri % 2], sins[ri % 2])

    descs_in = {0: start_in(0)}
    descs_out = {}
    for ri in range(_RPW):
        b = ri % 2
        descs_in[ri].wait()
        if ri + 1 < _RPW:
            if ri - 1 >= 0:
                descs_out[ri - 1].wait()
            descs_in[ri + 1] = start_in(ri + 1)
        compute(bufs[b])
        descs_out[ri] = pltpu.async_copy(
            bufs[b], o_hbm.at[pl.ds(r0 + ri, 1), :], souts[b])
    descs_out[_RPW - 2].wait()
    descs_out[_RPW - 1].wait()


def kernel(x):
    mesh = plsc.VectorSubcoreMesh(core_axis_name="c", subcore_axis_name="s")
    f = functools.partial(
        pl.kernel,
        mesh=mesh,
        out_type=jax.ShapeDtypeStruct((_M, _N), jnp.float32),
        scratch_types=[
            pltpu.VMEM((1, _N), jnp.float32),
            pltpu.VMEM((1, _N), jnp.float32),
            pltpu.SemaphoreType.DMA,
            pltpu.SemaphoreType.DMA,
            pltpu.SemaphoreType.DMA,
            pltpu.SemaphoreType.DMA,
        ],
        compiler_params=pltpu.CompilerParams(needs_layout_passes=False),
    )(_sc_body)
    return f(x)
